# Initial kernel scaffold; baseline (speedup 1.0000x reference)
import jax
import jax.numpy as jnp
from jax.experimental import pallas as pl


def kernel(x, edge_index, W1, b1, gamma1, beta1, W2, b2, W3, b3):
    raise NotImplementedError("write your pallas kernel here")


# speedup vs baseline: 11.1516x; 11.1516x over previous
import functools

import jax
import jax.numpy as jnp
from jax import lax
from jax.experimental import pallas as pl
from jax.experimental.pallas import tpu as pltpu
from jax.experimental.pallas import tpu_sc as plsc

N = 10000
NP = 10240
E = 160000
DH = 256
HH = 128
CH = 128
NCHUNK = E // CH
NC = 2
NS = 16
L = 16
F32 = jnp.float32

_mesh = plsc.VectorSubcoreMesh(
    core_axis_name="c", subcore_axis_name="s", num_cores=NC, num_subcores=NS)
_sc_params = pltpu.CompilerParams(needs_layout_passes=False)

W_CHUNKS = 39
S_CHUNKS = 78
H_CHUNKS = 39

NSL = NP // NS


def _sc_counts(dst, znp):
    @functools.partial(
        pl.kernel, mesh=_mesh, compiler_params=_sc_params,
        out_type=jax.ShapeDtypeStruct((NP,), F32),
        scratch_types=[
            pltpu.VMEM((CH,), jnp.int32),
            pltpu.VMEM((CH,), F32),
            pltpu.VMEM_SHARED((NP,), F32),
        ])
    def k(dst_hbm, z_hbm, cnt_hbm, didx, ones_v, cnt_sp):
        c = lax.axis_index("c")
        s = lax.axis_index("s")

        @pl.when(c == 0)
        def _():
            for i in range(CH // L):
                ones_v[pl.ds(i * L, L)] = jnp.ones((L,), F32)
            sl = pl.ds(pl.multiple_of(s * NSL, 8), NSL)
            pltpu.sync_copy(z_hbm.at[sl], cnt_sp.at[sl])
            plsc.subcore_barrier()

            def chunk(cid):
                e0 = pl.multiple_of(cid * CH, CH)
                pltpu.sync_copy(dst_hbm.at[pl.ds(e0, CH)], didx)
                pltpu.sync_copy(ones_v, cnt_sp.at[didx], add=True)

            @pl.loop(0, S_CHUNKS)
            def _(i):
                chunk(s * S_CHUNKS + i)

            @pl.when(s < 2)
            def _():
                chunk(16 * S_CHUNKS + s)

            plsc.subcore_barrier()
            pltpu.sync_copy(cnt_sp.at[sl], cnt_hbm.at[sl])

    return k(dst, znp)


def _sc_agg(hs, src, dst):
    @functools.partial(
        pl.kernel, mesh=_mesh, compiler_params=_sc_params,
        out_type=jax.ShapeDtypeStruct((2 * NP, HH), F32),
        scratch_types=[
            pltpu.VMEM((CH,), jnp.int32),
            pltpu.VMEM((CH,), jnp.int32),
            pltpu.VMEM((CH, HH), F32),
            pltpu.VMEM_SHARED((NP, HH), F32),
        ])
    def k(hs_hbm, src_hbm, dst_hbm, agg_hbm, sidx, didx, rows, acc_sp):
        c = lax.axis_index("c")
        s = lax.axis_index("s")
        off = c * NP
        rsl = pl.ds(pl.multiple_of(s * NSL, 8), NSL)
        gsl = pl.ds(pl.multiple_of(off + s * NSL, 8), NSL)
        pltpu.sync_copy(hs_hbm.at[gsl], acc_sp.at[rsl])
        plsc.subcore_barrier()

        def chunk(cid):
            e0 = pl.multiple_of(cid * CH, CH)
            pltpu.sync_copy(src_hbm.at[pl.ds(e0, CH)], sidx)
            pltpu.sync_copy(dst_hbm.at[pl.ds(e0, CH)], didx)
            for i in range(CH // L):
                sl = pl.ds(i * L, L)
                sidx[sl] = sidx[sl] + off
            pltpu.sync_copy(hs_hbm.at[sidx], rows)
            pltpu.sync_copy(rows, acc_sp.at[didx], add=True)

        @pl.loop(0, S_CHUNKS)
        def _(i):
            chunk(s * S_CHUNKS + i)

        @pl.when(s < 2)
        def _():
            chunk(16 * S_CHUNKS + s)

        plsc.subcore_barrier()
        pltpu.sync_copy(acc_sp.at[rsl], agg_hbm.at[gsl])

    return k(hs, src, dst)


def _sc_seg(p, q, dinv, src, dst, znp):
    @functools.partial(
        pl.kernel, mesh=_mesh, compiler_params=_sc_params,
        out_type=jax.ShapeDtypeStruct((NC, NP), F32),
        scratch_types=[
            pltpu.VMEM((CH,), jnp.int32),
            pltpu.VMEM((CH,), jnp.int32),
            pltpu.VMEM((CH,), F32),
            pltpu.VMEM((N,), F32),
            pltpu.VMEM((N,), F32),
            pltpu.VMEM((NP,), F32),
            pltpu.VMEM_SHARED((NP,), F32),
        ])
    def k(p_hbm, q_hbm, dinv_hbm, src_hbm, dst_hbm, z_hbm, segp_hbm,
          sidx, didx, tbuf, p_v, q_v, dinv_v, seg_sp):
        c = lax.axis_index("c")
        s = lax.axis_index("s")
        pltpu.sync_copy(p_hbm, p_v)
        pltpu.sync_copy(q_hbm, q_v)
        pltpu.sync_copy(dinv_hbm, dinv_v)
        sl = pl.ds(pl.multiple_of(s * NSL, 8), NSL)
        pltpu.sync_copy(z_hbm.at[sl], seg_sp.at[sl])
        plsc.subcore_barrier()

        def chunk(cid):
            e0 = pl.multiple_of(cid * CH, CH)
            pltpu.sync_copy(src_hbm.at[pl.ds(e0, CH)], sidx)
            pltpu.sync_copy(dst_hbm.at[pl.ds(e0, CH)], didx)
            for i in range(CH // L):
                vs = pl.ds(i * L, L)
                s16 = sidx[vs]
                d16 = didx[vs]
                pv = plsc.load_gather(p_v, [s16])
                qv = plsc.load_gather(q_v, [d16])
                dv = plsc.load_gather(dinv_v, [s16])
                tbuf[vs] = (pv + qv) * dv
            pltpu.sync_copy(tbuf, seg_sp.at[didx], add=True)

        base = c * 625 + s * H_CHUNKS

        @pl.loop(0, H_CHUNKS)
        def _(i):
            chunk(base + i)

        @pl.when(s == 0)
        def _():
            chunk(c * 625 + 624)

        plsc.subcore_barrier()
        pltpu.sync_copy(seg_sp.at[sl], segp_hbm.at[c, sl])

    return k(p, q, dinv, src, dst, znp)


def _sc_edge_out(p, q, dinv, segp, src, dst, b3b):
    @functools.partial(
        pl.kernel, mesh=_mesh, compiler_params=_sc_params,
        out_type=jax.ShapeDtypeStruct((E,), F32),
        scratch_types=[
            pltpu.VMEM((CH,), jnp.int32),
            pltpu.VMEM((CH,), jnp.int32),
            pltpu.VMEM((CH,), F32),
            pltpu.VMEM((N,), F32),
            pltpu.VMEM((N,), F32),
            pltpu.VMEM((NP,), F32),
            pltpu.VMEM((NP,), F32),
            pltpu.VMEM((NP,), F32),
            pltpu.VMEM((L,), F32),
        ])
    def k(p_hbm, q_hbm, dinv_hbm, segp_hbm, src_hbm, dst_hbm, b3_hbm,
          out_hbm, sidx, didx, obuf, p_v, q_v, dinv_v, sa_v, sb_v, b3_v):
        c = lax.axis_index("c")
        s = lax.axis_index("s")
        wid = s * NC + c
        pltpu.sync_copy(p_hbm, p_v)
        pltpu.sync_copy(q_hbm, q_v)
        pltpu.sync_copy(dinv_hbm, dinv_v)
        pltpu.sync_copy(segp_hbm.at[0], sa_v)
        pltpu.sync_copy(segp_hbm.at[1], sb_v)
        pltpu.sync_copy(b3_hbm, b3_v)
        bv = b3_v[...]

        @pl.loop(0, NP // L)
        def _(i):
            vs = pl.ds(pl.multiple_of(i * L, L), L)
            sa_v[vs] = sa_v[vs] + sb_v[vs]

        def chunk(cid):
            e0 = pl.multiple_of(cid * CH, CH)
            pltpu.sync_copy(src_hbm.at[pl.ds(e0, CH)], sidx)
            pltpu.sync_copy(dst_hbm.at[pl.ds(e0, CH)], didx)
            for i in range(CH // L):
                vs = pl.ds(i * L, L)
                s16 = sidx[vs]
                d16 = didx[vs]
                pv = plsc.load_gather(p_v, [s16])
                qv = plsc.load_gather(q_v, [d16])
                dv = plsc.load_gather(dinv_v, [s16])
                gv = plsc.load_gather(sa_v, [s16])
                hl = pv + qv
                val = dv * gv + dv * dv * hl + bv
                obuf[vs] = 1.0 / (1.0 + jnp.exp(-val))
            pltpu.sync_copy(obuf, out_hbm.at[pl.ds(e0, CH)])

        @pl.loop(0, W_CHUNKS)
        def _(i):
            chunk(wid * W_CHUNKS + i)

        @pl.when(wid < 2)
        def _():
            chunk(32 * W_CHUNKS + wid)

    return k(p, q, dinv, segp, src, dst, b3b)


def _tc_l1(x, W1, cnt, cnt_col):
    def body(x_ref, w_ref, cnt_ref, cc_ref, hs_ref, dinv_ref):
        dinv_ref[...] = lax.rsqrt(cnt_ref[...] + 1.0)
        dv = lax.rsqrt(cc_ref[0:N, :] + 1.0)
        h = jnp.dot(x_ref[...], w_ref[...], preferred_element_type=F32)
        hs = h * dv
        hs_ref[0:N, :] = hs[:, 0:HH]
        hs_ref[NP:NP + N, :] = hs[:, HH:]

    return pl.pallas_call(
        body,
        out_shape=(jax.ShapeDtypeStruct((2 * NP, HH), F32),
                   jax.ShapeDtypeStruct((NP,), F32)),
    )(x, W1, cnt, cnt_col)


def _tc_l2(agg1, cnt_col, b1r, g1r, be1r, W2):
    def body(a_ref, cc_ref, b_ref, g_ref, be_ref, w_ref, out_ref):
        dv = lax.rsqrt(cc_ref[0:N, :] + 1.0)

        def half(i):
            z = a_ref[i * NP:i * NP + N, :] * dv + b_ref[i:i + 1, :]
            mu = jnp.mean(z, axis=0, keepdims=True)
            var = jnp.mean((z - mu) ** 2, axis=0, keepdims=True)
            zn = (z - mu) * lax.rsqrt(var + 1e-5)
            return jnp.maximum(zn * g_ref[i:i + 1, :] + be_ref[i:i + 1, :],
                               0.0)

        h0 = half(0)
        h1 = half(1)
        hs = (jnp.dot(h0, w_ref[0:HH, :], preferred_element_type=F32) +
              jnp.dot(h1, w_ref[HH:, :], preferred_element_type=F32))
        hs = hs * dv
        out_ref[0:N, :] = hs[:, 0:HH]
        out_ref[NP:NP + N, :] = hs[:, HH:]

    return pl.pallas_call(
        body,
        out_shape=jax.ShapeDtypeStruct((2 * NP, HH), F32),
    )(agg1, cnt_col, b1r, g1r, be1r, W2)


def _tc_l3(agg2, cnt_col, b2r, Wpq):
    def body(a_ref, cc_ref, b_ref, w_ref, pq_ref):
        dv = lax.rsqrt(cc_ref[0:N, :] + 1.0)
        h0 = jnp.maximum(a_ref[0:N, :] * dv + b_ref[0:1, :], 0.0)
        h1 = jnp.maximum(a_ref[NP:NP + N, :] * dv + b_ref[1:2, :], 0.0)
        pq_ref[...] = (
            jnp.dot(h0, w_ref[0:HH, :], preferred_element_type=F32) +
            jnp.dot(h1, w_ref[HH:, :], preferred_element_type=F32))

    return pl.pallas_call(
        body,
        out_shape=jax.ShapeDtypeStruct((N, 2), F32),
    )(agg2, cnt_col, b2r, Wpq)


def kernel(x, edge_index, W1, b1, gamma1, beta1, W2, b2, W3, b3):
    src = edge_index[0].astype(jnp.int32)
    dst = edge_index[1].astype(jnp.int32)
    znp = jnp.zeros((NP,), F32)

    cnt = _sc_counts(dst, znp)
    cnt_col = cnt[:N].reshape(N, 1)

    hs1, dinv = _tc_l1(x, W1, cnt, cnt_col)
    agg1 = _sc_agg(hs1, src, dst)
    hs2 = _tc_l2(agg1, cnt_col,
                 b1.reshape(2, HH), gamma1.reshape(2, HH),
                 beta1.reshape(2, HH), W2)
    agg2 = _sc_agg(hs2, src, dst)
    Wpq = jnp.concatenate([W3[:DH], W3[DH:]], axis=1)
    pq = _tc_l3(agg2, cnt_col, b2.reshape(2, HH), Wpq)

    p = pq[:, 0]
    q = pq[:, 1]
    segp = _sc_seg(p, q, dinv, src, dst, znp)
    b3b = jnp.broadcast_to(b3, (L,)).astype(F32)
    out = _sc_edge_out(p, q, dinv, segp, src, dst, b3b)
    return out.reshape(E, 1)


# speedup vs baseline: 18.3998x; 1.6500x over previous
import functools

import jax
import jax.numpy as jnp
from jax import lax
from jax.experimental import pallas as pl
from jax.experimental.pallas import tpu as pltpu
from jax.experimental.pallas import tpu_sc as plsc

N = 10000
NP = 10240
E = 160000
DH = 256
HH = 128
CH = 128
NCHUNK = E // CH
NCHP = 1280
NC = 2
NS = 16
L = 16
F32 = jnp.float32

_mesh = plsc.VectorSubcoreMesh(
    core_axis_name="c", subcore_axis_name="s", num_cores=NC, num_subcores=NS)
_sc_params = pltpu.CompilerParams(needs_layout_passes=False)

W_CHUNKS = 39
S_CHUNKS = 78
NSL = NP // NS


def _aligned_base(start):
    b8 = pl.multiple_of((start // 8) * 8, 8)
    return b8, start - b8


def _sc_counts(ei3, znp):
    HNP = NP // 2

    @functools.partial(
        pl.kernel, mesh=_mesh, compiler_params=_sc_params,
        out_type=jax.ShapeDtypeStruct((NP,), F32),
        scratch_types=[
            pltpu.VMEM((88, CH), jnp.int32),
            pltpu.VMEM((CH,), F32),
            pltpu.SemaphoreType.DMA,
            pltpu.VMEM_SHARED((NP + 128,), F32),
        ])
    def k(ei_hbm, z_hbm, cnt_hbm, didx, ones_v, sem, cnt_sp):
        c = lax.axis_index("c")
        s = lax.axis_index("s")
        lo = c * HNP
        hi = lo + HNP
        for i in range(CH // L):
            ones_v[pl.ds(i * L, L)] = jnp.ones((L,), F32)
        sl = pl.ds(pl.multiple_of(s * NSL, 8), NSL)
        pltpu.sync_copy(z_hbm.at[sl], cnt_sp.at[sl])
        b8, r = _aligned_base(s * S_CHUNKS)
        pltpu.sync_copy(ei_hbm.at[1, pl.ds(b8, 88)], didx)
        plsc.subcore_barrier()

        def mask_row(j):
            for i in range(CH // L):
                vs = pl.ds(i * L, L)
                v = didx[j, vs]
                inr = (v >= lo) & (v < hi)
                didx[j, vs] = jnp.where(inr, v, NP)

        @pl.loop(0, S_CHUNKS)
        def _(j):
            mask_row(r + j)

        @pl.loop(0, S_CHUNKS)
        def _(j):
            pltpu.async_copy(ones_v, cnt_sp.at[didx.at[r + j]], sem, add=True)

        @pl.loop(0, S_CHUNKS)
        def _(j):
            pltpu.make_async_copy(ones_v, cnt_sp.at[didx.at[r]], sem).wait()

        @pl.when(s < 2)
        def _():
            pltpu.sync_copy(ei_hbm.at[1, pl.ds(1248, 8)], didx.at[pl.ds(0, 8)])
            mask_row(s)
            pltpu.async_copy(ones_v, cnt_sp.at[didx.at[s]], sem, add=True)
            pltpu.make_async_copy(ones_v, cnt_sp.at[didx.at[s]], sem).wait()

        plsc.subcore_barrier()

        @pl.when(s < 8)
        def _():
            wsl = pl.ds(pl.multiple_of(lo + s * NSL, 8), NSL)
            pltpu.sync_copy(cnt_sp.at[wsl], cnt_hbm.at[wsl])

    return k(ei3, znp)


def _sc_agg(hs3, ei3):
    @functools.partial(
        pl.kernel, mesh=_mesh, compiler_params=_sc_params,
        out_type=jax.ShapeDtypeStruct((NC, NP, HH), F32),
        scratch_types=[
            pltpu.VMEM((48, CH), jnp.int32),
            pltpu.VMEM((48, CH), jnp.int32),
            pltpu.VMEM((2, CH, HH), F32),
            [pltpu.SemaphoreType.DMA] * 2,
            pltpu.VMEM_SHARED((NP, HH), F32),
        ])
    def k(hs_hbm, ei_hbm, agg_hbm, sidx, didx, rows, gsems, acc_sp):
        c = lax.axis_index("c")
        s = lax.axis_index("s")
        rsl = pl.ds(pl.multiple_of(s * NSL, 8), NSL)
        pltpu.sync_copy(hs_hbm.at[c, rsl], acc_sp.at[rsl])
        plsc.subcore_barrier()

        def sg(j, b):
            pltpu.async_copy(hs_hbm.at[c].at[sidx.at[j]], rows.at[b],
                             gsems[b])

        def wg(j, b):
            pltpu.make_async_copy(hs_hbm.at[c].at[sidx.at[j]],
                                  rows.at[b], gsems[b]).wait()

        def sc(j, b):
            pltpu.sync_copy(rows.at[b], acc_sp.at[didx.at[j]], add=True)

        def phase(start, n):
            b8, r = _aligned_base(start)
            pltpu.sync_copy(ei_hbm.at[0, pl.ds(b8, 48)], sidx)
            pltpu.sync_copy(ei_hbm.at[1, pl.ds(b8, 48)], didx)
            sg(r, 0)

            @pl.loop(0, n // 2)
            def _(t):
                for u in range(2):
                    j = r + t * 2 + u
                    nxt = jnp.minimum(j + 1, r + n - 1)
                    sg(nxt, 1 - u)
                    wg(j, u)
                    sc(j, u)

            wg(r, 0)

        phase(s * S_CHUNKS, 40)
        phase(s * S_CHUNKS + 40, 38)

        @pl.when(s < 2)
        def _():
            pltpu.sync_copy(ei_hbm.at[0, pl.ds(1248, 8)],
                            sidx.at[pl.ds(0, 8)])
            pltpu.sync_copy(ei_hbm.at[1, pl.ds(1248, 8)],
                            didx.at[pl.ds(0, 8)])
            sg(s, 0)
            wg(s, 0)
            sc(s, 0)

        plsc.subcore_barrier()
        pltpu.sync_copy(acc_sp.at[rsl], agg_hbm.at[c, rsl])

    return k(hs3, ei3)


def _sc_seg(p, q, dinv, ei3, znp):
    @functools.partial(
        pl.kernel, mesh=_mesh, compiler_params=_sc_params,
        out_type=jax.ShapeDtypeStruct((NC, NP), F32),
        scratch_types=[
            pltpu.VMEM((48, CH), jnp.int32),
            pltpu.VMEM((48, CH), jnp.int32),
            pltpu.VMEM((48, CH), F32),
            pltpu.VMEM((N,), F32),
            pltpu.VMEM((N,), F32),
            pltpu.VMEM((NP,), F32),
            pltpu.SemaphoreType.DMA,
            pltpu.VMEM_SHARED((NP,), F32),
        ])
    def k(p_hbm, q_hbm, dinv_hbm, ei_hbm, z_hbm, segp_hbm,
          sidx, didx, tbuf, p_v, q_v, dinv_v, sem, seg_sp):
        c = lax.axis_index("c")
        s = lax.axis_index("s")
        pltpu.sync_copy(p_hbm, p_v)
        pltpu.sync_copy(q_hbm, q_v)
        pltpu.sync_copy(dinv_hbm, dinv_v)
        sl = pl.ds(pl.multiple_of(s * NSL, 8), NSL)
        pltpu.sync_copy(z_hbm.at[sl], seg_sp.at[sl])
        b8, r = _aligned_base(c * 625 + s * W_CHUNKS)
        pltpu.sync_copy(ei_hbm.at[0, pl.ds(b8, 48)], sidx)
        pltpu.sync_copy(ei_hbm.at[1, pl.ds(b8, 48)], didx)
        plsc.subcore_barrier()

        def compute_row(j):
            for i in range(CH // L):
                vs = pl.ds(i * L, L)
                s16 = sidx[j, vs]
                d16 = didx[j, vs]
                pv = plsc.load_gather(p_v, [s16])
                qv = plsc.load_gather(q_v, [d16])
                dv = plsc.load_gather(dinv_v, [s16])
                tbuf[j, vs] = (pv + qv) * dv

        @pl.loop(0, W_CHUNKS)
        def _(j):
            compute_row(r + j)

        @pl.loop(0, W_CHUNKS)
        def _(j):
            pltpu.async_copy(tbuf.at[r + j], seg_sp.at[didx.at[r + j]],
                             sem, add=True)

        @pl.loop(0, W_CHUNKS)
        def _(j):
            pltpu.make_async_copy(tbuf.at[r], seg_sp.at[didx.at[r]],
                                  sem).wait()

        @pl.when(s == 0)
        def _():
            xb8, xr = _aligned_base(c * 625 + 624)
            pltpu.sync_copy(ei_hbm.at[0, pl.ds(xb8, 8)], sidx.at[pl.ds(0, 8)])
            pltpu.sync_copy(ei_hbm.at[1, pl.ds(xb8, 8)], didx.at[pl.ds(0, 8)])
            compute_row(xr)
            pltpu.async_copy(tbuf.at[xr], seg_sp.at[didx.at[xr]], sem,
                             add=True)
            pltpu.make_async_copy(tbuf.at[xr], seg_sp.at[didx.at[xr]],
                                  sem).wait()

        plsc.subcore_barrier()
        pltpu.sync_copy(seg_sp.at[sl], segp_hbm.at[c, sl])

    return k(p, q, dinv, ei3, znp)


def _sc_edge_out(p, q, dinv, segp, ei3, b3b):
    @functools.partial(
        pl.kernel, mesh=_mesh, compiler_params=_sc_params,
        out_type=jax.ShapeDtypeStruct((E,), F32),
        scratch_types=[
            pltpu.VMEM((48, CH), jnp.int32),
            pltpu.VMEM((48, CH), jnp.int32),
            pltpu.VMEM((W_CHUNKS * CH,), F32),
            pltpu.VMEM((CH,), F32),
            pltpu.VMEM((N,), F32),
            pltpu.VMEM((N,), F32),
            pltpu.VMEM((NP,), F32),
            pltpu.VMEM((NP,), F32),
            pltpu.VMEM((NP,), F32),
            pltpu.VMEM((L,), F32),
        ])
    def k(p_hbm, q_hbm, dinv_hbm, segp_hbm, ei_hbm, b3_hbm, out_hbm,
          sidx, didx, obuf, obx, p_v, q_v, dinv_v, sa_v, sb_v, b3_v):
        c = lax.axis_index("c")
        s = lax.axis_index("s")
        wid = s * NC + c
        pltpu.sync_copy(p_hbm, p_v)
        pltpu.sync_copy(q_hbm, q_v)
        pltpu.sync_copy(dinv_hbm, dinv_v)
        pltpu.sync_copy(segp_hbm.at[0], sa_v)
        pltpu.sync_copy(segp_hbm.at[1], sb_v)
        pltpu.sync_copy(b3_hbm, b3_v)
        bv = b3_v[...]
        b8, r = _aligned_base(wid * W_CHUNKS)
        pltpu.sync_copy(ei_hbm.at[0, pl.ds(b8, 48)], sidx)
        pltpu.sync_copy(ei_hbm.at[1, pl.ds(b8, 48)], didx)

        @pl.loop(0, NP // L)
        def _(i):
            vs = pl.ds(pl.multiple_of(i * L, L), L)
            sa_v[vs] = sa_v[vs] + sb_v[vs]

        def compute_row(j, out_ref, ob):
            for i in range(CH // L):
                vs = pl.ds(i * L, L)
                s16 = sidx[j, vs]
                d16 = didx[j, vs]
                pv = plsc.load_gather(p_v, [s16])
                qv = plsc.load_gather(q_v, [d16])
                dv = plsc.load_gather(dinv_v, [s16])
                gv = plsc.load_gather(sa_v, [s16])
                hl = pv + qv
                val = dv * gv + dv * dv * hl + bv
                out_ref[pl.ds(ob + i * L, L)] = 1.0 / (1.0 + jnp.exp(-val))

        @pl.loop(0, W_CHUNKS)
        def _(j):
            compute_row(r + j, obuf, j * CH)

        pltpu.sync_copy(
            obuf, out_hbm.at[pl.ds(pl.multiple_of(wid * (W_CHUNKS * CH), 8),
                                   W_CHUNKS * CH)])

        @pl.when(wid < 2)
        def _():
            pltpu.sync_copy(ei_hbm.at[0, pl.ds(1248, 8)], sidx.at[pl.ds(0, 8)])
            pltpu.sync_copy(ei_hbm.at[1, pl.ds(1248, 8)], didx.at[pl.ds(0, 8)])
            compute_row(wid, obx, 0)
            pltpu.sync_copy(
                obx, out_hbm.at[pl.ds((1248 + wid) * CH, CH)])

    return k(p, q, dinv, segp, ei3, b3b)


def _tc_l1(x, W1, cnt, cnt_col):
    def body(x_ref, w_ref, cnt_ref, cc_ref, hs_ref, dinv_ref):
        dinv_ref[...] = lax.rsqrt(cnt_ref[...] + 1.0)
        dv = lax.rsqrt(cc_ref[0:N, :] + 1.0)
        h = jnp.dot(x_ref[...], w_ref[...], preferred_element_type=F32)
        hs = h * dv
        hs_ref[0, 0:N, :] = hs[:, 0:HH]
        hs_ref[1, 0:N, :] = hs[:, HH:]

    return pl.pallas_call(
        body,
        out_shape=(jax.ShapeDtypeStruct((NC, NP, HH), F32),
                   jax.ShapeDtypeStruct((NP,), F32)),
    )(x, W1, cnt, cnt_col)


def _tc_l2(agg1, cnt_col, b1r, g1r, be1r, W2):
    def body(a_ref, cc_ref, b_ref, g_ref, be_ref, w_ref, out_ref):
        dv = lax.rsqrt(cc_ref[0:N, :] + 1.0)

        def half(i):
            z = a_ref[i, 0:N, :] * dv + b_ref[i:i + 1, :]
            mu = jnp.mean(z, axis=0, keepdims=True)
            var = jnp.mean((z - mu) ** 2, axis=0, keepdims=True)
            zn = (z - mu) * lax.rsqrt(var + 1e-5)
            return jnp.maximum(zn * g_ref[i:i + 1, :] + be_ref[i:i + 1, :],
                               0.0)

        h0 = half(0)
        h1 = half(1)
        hs = (jnp.dot(h0, w_ref[0:HH, :], preferred_element_type=F32) +
              jnp.dot(h1, w_ref[HH:, :], preferred_element_type=F32))
        hs = hs * dv
        out_ref[0, 0:N, :] = hs[:, 0:HH]
        out_ref[1, 0:N, :] = hs[:, HH:]

    return pl.pallas_call(
        body,
        out_shape=jax.ShapeDtypeStruct((NC, NP, HH), F32),
    )(agg1, cnt_col, b1r, g1r, be1r, W2)


def _tc_l3(agg2, cnt_col, b2r, Wpq):
    def body(a_ref, cc_ref, b_ref, w_ref, pq_ref):
        dv = lax.rsqrt(cc_ref[0:N, :] + 1.0)
        h0 = jnp.maximum(a_ref[0, 0:N, :] * dv + b_ref[0:1, :], 0.0)
        h1 = jnp.maximum(a_ref[1, 0:N, :] * dv + b_ref[1:2, :], 0.0)
        pq_ref[...] = (
            jnp.dot(h0, w_ref[0:HH, :], preferred_element_type=F32) +
            jnp.dot(h1, w_ref[HH:, :], preferred_element_type=F32))

    return pl.pallas_call(
        body,
        out_shape=jax.ShapeDtypeStruct((N, 2), F32),
    )(agg2, cnt_col, b2r, Wpq)


def kernel(x, edge_index, W1, b1, gamma1, beta1, W2, b2, W3, b3):
    ei = edge_index.astype(jnp.int32)
    ei3 = jnp.pad(ei, ((0, 0), (0, NCHP * CH - E))).reshape(2, NCHP, CH)
    znp = jnp.zeros((NP,), F32)

    cnt = _sc_counts(ei3, znp)
    cnt_col = cnt[:N].reshape(N, 1)

    hs1, dinv = _tc_l1(x, W1, cnt, cnt_col)
    agg1 = _sc_agg(hs1, ei3)
    hs2 = _tc_l2(agg1, cnt_col,
                 b1.reshape(2, HH), gamma1.reshape(2, HH),
                 beta1.reshape(2, HH), W2)
    agg2 = _sc_agg(hs2, ei3)
    Wpq = jnp.concatenate([W3[:DH], W3[DH:]], axis=1)
    pq = _tc_l3(agg2, cnt_col, b2.reshape(2, HH), Wpq)

    p = pq[:, 0]
    q = pq[:, 1]
    segp = _sc_seg(p, q, dinv, ei3, znp)
    b3b = jnp.broadcast_to(b3, (L,)).astype(F32)
    out = _sc_edge_out(p, q, dinv, segp, ei3, b3b)
    return out.reshape(E, 1)
